# baseline (speedup 1.0000x reference)
import functools

import jax
import jax.numpy as jnp
from jax import lax
from jax.experimental import pallas as pl
from jax.experimental.pallas import tpu as pltpu
from jax.experimental.pallas import tpu_sc as plsc

NC = 2
NS = 16
NW = NC * NS


@functools.partial(jax.jit, static_argnums=(2, 3, 4, 5))
def _sc_gather(tab, idx, F, B, C, D):
  mesh = plsc.VectorSubcoreMesh(
      core_axis_name="c", subcore_axis_name="s", num_cores=NC, num_subcores=NS)

  @functools.partial(
      pl.kernel,
      out_type=jax.ShapeDtypeStruct((B, F * D), jnp.float32),
      mesh=mesh,
      compiler_params=pltpu.CompilerParams(use_tc_tiling_on_sc=False),
      scratch_types=[
          pltpu.VMEM((F * C,), jnp.int32),
          pltpu.VMEM((C, D), jnp.float32),
          pltpu.SemaphoreType.DMA,
      ],
  )
  def k(tab_hbm, idx_hbm, out_hbm, idx_v, rows_v, sem):
    wid = lax.axis_index("s") * NC + lax.axis_index("c")
    base = wid * C
    pltpu.sync_copy(idx_hbm.at[pl.ds(wid * (F * C), F * C)], idx_v)

    def per_field(f, carry):
      pltpu.async_copy(tab_hbm.at[idx_v.at[pl.ds(f * C, C)]], rows_v, sem).wait()
      pltpu.sync_copy(rows_v, out_hbm.at[pl.ds(base, C), pl.ds(f * D, D)])
      return carry

    lax.fori_loop(0, F, per_field, 0)

  return k(tab, idx)


def _fm_body(raw_ref, wt_ref, e_ref, kt_ref, wrow_ref, bias_ref, out_ref):
  wexp = jnp.dot(wt_ref[...], e_ref[...], preferred_element_type=jnp.float32)
  u = raw_ref[...] * wexp
  m2 = jnp.dot(u.astype(jnp.bfloat16), kt_ref[...],
               preferred_element_type=jnp.float32)
  t = u * (m2 + wrow_ref[...])
  out_ref[...] = jnp.sum(t, axis=1) + bias_ref[0]


@jax.jit
def _fm(raw, wt, E, KT, Wrow, bias):
  B, FD = raw.shape
  F = wt.shape[1]
  Bc = 512
  grid = (B // Bc,)
  return pl.pallas_call(
      _fm_body,
      grid=grid,
      in_specs=[
          pl.BlockSpec((Bc, FD), lambda i: (i, 0)),
          pl.BlockSpec((Bc, F), lambda i: (i, 0)),
          pl.BlockSpec((F, FD), lambda i: (0, 0)),
          pl.BlockSpec((FD, FD), lambda i: (0, 0)),
          pl.BlockSpec((1, FD), lambda i: (0, 0)),
          pl.BlockSpec(memory_space=pltpu.SMEM),
      ],
      out_specs=pl.BlockSpec((Bc,), lambda i: (i,)),
      out_shape=jax.ShapeDtypeStruct((B,), jnp.float32),
  )(raw, wt, E, KT, Wrow, bias)


def kernel(idxs, weights, tables, W, r, bias):
  F, B, L = idxs.shape
  V, D = tables.shape[1], tables.shape[2]
  C = B // NW

  flat = idxs[..., 0].astype(jnp.int32) + (
      jnp.arange(F, dtype=jnp.int32) * V)[:, None]
  idxT = flat.reshape(F, NW, C).transpose(1, 0, 2).reshape(-1)
  tabF = tables.reshape(F * V, D)

  raw = _sc_gather(tabF, idxT, F, B, C, D)

  wt = weights[..., 0].T
  rt = jnp.triu(r, 1)
  E = jnp.repeat(jnp.eye(F, dtype=jnp.float32), D, axis=1)
  KT = jnp.kron(rt.T, jnp.eye(D, dtype=jnp.float32)).astype(jnp.bfloat16)
  return _fm(raw, wt, E, KT, W, bias)
